# Initial kernel scaffold; baseline (speedup 1.0000x reference)
import jax
import jax.numpy as jnp
from jax.experimental import pallas as pl


def kernel(t, cycleQueue):
    raise NotImplementedError("write your pallas kernel here")


# speedup vs baseline: 1.3510x; 1.3510x over previous
import jax
import jax.numpy as jnp
from jax.experimental import pallas as pl
from jax.experimental.pallas import tpu as pltpu
from jax.experimental.pallas import tpu_sc as plsc

W = 168
D = 512
B = 16384
NW = 32
BPW = B // NW
CH = 64
LANES = 16


def kernel(t, cycleQueue):
    t32 = t.astype(jnp.int32)
    mesh = plsc.VectorSubcoreMesh(core_axis_name="core", subcore_axis_name="subcore")

    @pl.kernel(
        out_type=jax.ShapeDtypeStruct((B, D), jnp.float32),
        mesh=mesh,
        scratch_types=[
            pltpu.VMEM((BPW,), jnp.int32),
            pltpu.VMEM((CH, D), jnp.float32),
            pltpu.SemaphoreType.DMA,
        ],
    )
    def run(t_hbm, q_hbm, o_hbm, idx_v, rows_v, sem):
        wid = jax.lax.axis_index("subcore") * 2 + jax.lax.axis_index("core")
        base = wid * BPW
        pltpu.sync_copy(t_hbm.at[pl.ds(base, BPW)], idx_v)

        @pl.loop(0, BPW, step=LANES)
        def _(c):
            sl = pl.ds(c, LANES)
            idx_v.at[sl][...] = jax.lax.rem(idx_v.at[sl][...], jnp.int32(W))

        @pl.loop(0, BPW, step=CH)
        def _(c):
            pltpu.async_copy(q_hbm.at[idx_v.at[pl.ds(c, CH)]], rows_v, sem).wait()
            pltpu.sync_copy(rows_v, o_hbm.at[pl.ds(base + c, CH)])

    return run(t32, cycleQueue)
